# baseline (speedup 1.0000x reference)
import functools
import math

import jax
import jax.numpy as jnp
from jax import lax
from jax.experimental import pallas as pl
from jax.experimental.pallas import tpu as pltpu
from jax.experimental.pallas import tpu_sc as plsc

D = 1024
H = 16
DH = 64
DFF = 4096
B = 4
T = 8192
K = 1024

TBLK = 2048
JCH = 256
FBLK = 1024

_NW = 32
_RPW = (B * K) // _NW
_GCH = 64


def _score_copy_body(x_ref, wr_ref, out_ref, s_ref):
    xb = x_ref[0]
    out_ref[0] = xb
    s = jnp.dot(xb, wr_ref[0], preferred_element_type=jnp.float32)
    s_ref[0, 0] = jax.nn.sigmoid(s)


def _score_copy(x, w_router):
    return pl.pallas_call(
        _score_copy_body,
        grid=(B, T // TBLK),
        in_specs=[
            pl.BlockSpec((1, TBLK, D), lambda b, t: (b, t, 0)),
            pl.BlockSpec((1, D), lambda b, t: (0, 0)),
        ],
        out_specs=[
            pl.BlockSpec((1, TBLK, D), lambda b, t: (b, t, 0)),
            pl.BlockSpec((1, 1, TBLK), lambda b, t: (b, 0, t)),
        ],
        out_shape=[
            jax.ShapeDtypeStruct((B, T, D), jnp.float32),
            jax.ShapeDtypeStruct((B, 1, T), jnp.float32),
        ],
    )(x, w_router)


def _select_body(s_ref, idx_ref, aux_ref):
    b = pl.program_id(0)
    c = pl.program_id(1)
    s = s_ref[:, 0, :]
    bits = lax.bitcast_convert_type(s, jnp.int32)

    lo = jnp.full((B, 1), -1, jnp.int32)
    hi = jnp.full((B, 1), 0x3F800000, jnp.int32)

    def bis(_, lh):
        lo, hi = lh
        mid = lax.div(lo + hi, 2)
        cnt = jnp.sum((bits > mid).astype(jnp.int32), axis=1, keepdims=True)
        take_hi = cnt < K
        return jnp.where(take_hi, lo, mid), jnp.where(take_hi, mid, hi)

    lo, hi = lax.fori_loop(0, 31, bis, (lo, hi))
    thr = hi
    gt = bits > thr
    eqm = bits == thr
    cgt = jnp.sum(gt.astype(jnp.int32), axis=1, keepdims=True)
    tneed = K - cgt
    pos = lax.broadcasted_iota(jnp.int32, (B, T), 1)

    lo2 = jnp.zeros((B, 1), jnp.int32)
    hi2 = jnp.full((B, 1), T, jnp.int32)

    def bis2(_, lh):
        lo, hi = lh
        mid = lax.div(lo + hi, 2)
        cnt = jnp.sum((eqm & (pos < mid)).astype(jnp.int32), axis=1,
                      keepdims=True)
        ok = cnt >= tneed
        return jnp.where(ok, lo, mid), jnp.where(ok, mid, hi)

    lo2, hi2 = lax.fori_loop(0, 13, bis2, (lo2, hi2))
    mask = gt | (eqm & (pos < hi2))
    mf = mask.astype(jnp.float32)

    mfr = mf.reshape(B * 64, 128)
    i128 = lax.broadcasted_iota(jnp.int32, (128, 128), 0)
    j128 = lax.broadcasted_iota(jnp.int32, (128, 128), 1)
    tri = (i128 <= j128).astype(jnp.float32)
    csum = jnp.dot(mfr, tri, preferred_element_type=jnp.float32)
    csum = csum.reshape(B, 64, 128)
    ssum = csum[:, :, 127]
    i64 = lax.broadcasted_iota(jnp.int32, (64, 64), 0)
    j64 = lax.broadcasted_iota(jnp.int32, (64, 64), 1)
    stri = (i64 < j64).astype(jnp.float32)
    off = jnp.dot(ssum, stri, preferred_element_type=jnp.float32)
    p = (csum + off[:, :, None]).reshape(B, T)

    rowm = lax.broadcasted_iota(jnp.int32, (B, T), 0) == b
    p_b = jnp.sum(jnp.where(rowm, p, 0.0), axis=0)
    m_b = jnp.sum(jnp.where(rowm & mask, 1.0, 0.0), axis=0)

    jvals = (lax.broadcasted_iota(jnp.int32, (1, JCH), 1) + c * JCH + 1
             ).astype(jnp.float32)
    onehot = ((p_b[:, None] == jvals) & (m_b[:, None] > 0.5)
              ).astype(jnp.float32)
    tv = lax.broadcasted_iota(jnp.int32, (1, T), 1).astype(jnp.float32)
    vals = jnp.dot(tv, onehot, preferred_element_type=jnp.float32)
    idx_ref[0, 0, :] = (vals[0] + b.astype(jnp.float32) * T).astype(jnp.int32)

    rs = jnp.sum(s, axis=1) / T
    mu = jnp.mean(rs)
    aux_ref[...] = (jnp.sum((rs - mu) ** 2) / (B - 1)).reshape(1, 1)


def _select(scores):
    return pl.pallas_call(
        _select_body,
        grid=(B, K // JCH),
        in_specs=[pl.BlockSpec((B, 1, T), lambda b, c: (0, 0, 0))],
        out_specs=[
            pl.BlockSpec((1, 1, JCH), lambda b, c: (b, 0, c)),
            pl.BlockSpec((1, 1), lambda b, c: (0, 0)),
        ],
        out_shape=[
            jax.ShapeDtypeStruct((B, 1, K), jnp.int32),
            jax.ShapeDtypeStruct((1, 1), jnp.float32),
        ],
    )(scores)


@functools.cache
def _make_sc_gather():
    mesh = plsc.VectorSubcoreMesh(core_axis_name="c", subcore_axis_name="s")

    @functools.partial(
        pl.kernel,
        out_type=jax.ShapeDtypeStruct((B * K, D), jnp.float32),
        mesh=mesh,
        scratch_types=[
            pltpu.VMEM((_GCH,), jnp.int32),
            pltpu.VMEM((_GCH, D), jnp.float32),
            pltpu.SemaphoreType.DMA,
        ],
    )
    def _sc_gather(x_hbm, idx_hbm, out_hbm, idx_v, rows_v, sem):
        wid = lax.axis_index("s") * 2 + lax.axis_index("c")
        base = wid * _RPW
        for ch in range(_RPW // _GCH):
            off = base + ch * _GCH
            pltpu.sync_copy(idx_hbm.at[pl.ds(off, _GCH)], idx_v)
            pltpu.async_copy(x_hbm.at[idx_v], rows_v, sem).wait()
            pltpu.sync_copy(rows_v, out_hbm.at[pl.ds(off, _GCH)])

    return _sc_gather


def _attn_body(xs_ref, wq_ref, wk_ref, wv_ref, wo_ref, g_ref, b_ref,
               out_ref, nrm_ref):
    h = pl.program_id(1)
    xb = xs_ref[0]

    @pl.when(h == 0)
    def _():
        mu = jnp.mean(xb, axis=1, keepdims=True)
        var = jnp.mean((xb - mu) ** 2, axis=1, keepdims=True)
        nrm_ref[...] = ((xb - mu) * lax.rsqrt(var + 1e-5) * g_ref[0]
                        + b_ref[0])

    normed = nrm_ref[...]
    cdim = (((1,), (1,)), ((), ()))
    q = lax.dot_general(normed, wq_ref[...], cdim,
                        preferred_element_type=jnp.float32)
    k = lax.dot_general(normed, wk_ref[...], cdim,
                        preferred_element_type=jnp.float32)
    v = lax.dot_general(normed, wv_ref[...], cdim,
                        preferred_element_type=jnp.float32)
    att = lax.dot_general(q, k, cdim,
                          preferred_element_type=jnp.float32) * (1.0 / 8.0)
    att = att - jnp.max(att, axis=1, keepdims=True)
    att = jnp.exp(att)
    att = att / jnp.sum(att, axis=1, keepdims=True)
    o = lax.dot_general(att, v, (((1,), (0,)), ((), ())),
                        preferred_element_type=jnp.float32)
    contrib = lax.dot_general(o, wo_ref[...], (((1,), (0,)), ((), ())),
                              preferred_element_type=jnp.float32)

    @pl.when(h == 0)
    def _():
        out_ref[0] = xb + contrib

    @pl.when(h != 0)
    def _():
        out_ref[0] = out_ref[0] + contrib


def _attn(x_sel, wq, wk, wv, wo, g1, b1):
    return pl.pallas_call(
        _attn_body,
        grid=(B, H),
        in_specs=[
            pl.BlockSpec((1, K, D), lambda b, h: (b, 0, 0)),
            pl.BlockSpec((DH, D), lambda b, h: (h, 0)),
            pl.BlockSpec((DH, D), lambda b, h: (h, 0)),
            pl.BlockSpec((DH, D), lambda b, h: (h, 0)),
            pl.BlockSpec((DH, D), lambda b, h: (h, 0)),
            pl.BlockSpec((1, D), lambda b, h: (0, 0)),
            pl.BlockSpec((1, D), lambda b, h: (0, 0)),
        ],
        out_specs=pl.BlockSpec((1, K, D), lambda b, h: (b, 0, 0)),
        out_shape=jax.ShapeDtypeStruct((B, K, D), jnp.float32),
        scratch_shapes=[pltpu.VMEM((K, D), jnp.float32)],
    )(x_sel, wq, wk, wv, wo, g1, b1)


def _ffn_body(x_ref, w1_ref, w2_ref, w3_ref, g_ref, b_ref, out_ref, h_ref):
    f = pl.program_id(1)
    xb = x_ref[0]

    @pl.when(f == 0)
    def _():
        mu = jnp.mean(xb, axis=1, keepdims=True)
        var = jnp.mean((xb - mu) ** 2, axis=1, keepdims=True)
        h_ref[...] = ((xb - mu) * lax.rsqrt(var + 1e-5) * g_ref[0]
                      + b_ref[0])

    hh = h_ref[...]
    cdim = (((1,), (1,)), ((), ()))
    a = lax.dot_general(hh, w1_ref[...], cdim,
                        preferred_element_type=jnp.float32)
    bb = lax.dot_general(hh, w2_ref[...], cdim,
                         preferred_element_type=jnp.float32)
    gg = (a * jax.nn.sigmoid(a)) * bb
    contrib = lax.dot_general(gg, w3_ref[...], cdim,
                              preferred_element_type=jnp.float32)

    @pl.when(f == 0)
    def _():
        out_ref[0] = xb + contrib

    @pl.when(f != 0)
    def _():
        out_ref[0] = out_ref[0] + contrib


def _ffn(x1, w1, w2, w3, g2, b2):
    return pl.pallas_call(
        _ffn_body,
        grid=(B, DFF // FBLK),
        in_specs=[
            pl.BlockSpec((1, K, D), lambda b, f: (b, 0, 0)),
            pl.BlockSpec((FBLK, D), lambda b, f: (f, 0)),
            pl.BlockSpec((FBLK, D), lambda b, f: (f, 0)),
            pl.BlockSpec((D, FBLK), lambda b, f: (0, f)),
            pl.BlockSpec((1, D), lambda b, f: (0, 0)),
            pl.BlockSpec((1, D), lambda b, f: (0, 0)),
        ],
        out_specs=pl.BlockSpec((1, K, D), lambda b, f: (b, 0, 0)),
        out_shape=jax.ShapeDtypeStruct((B, K, D), jnp.float32),
        scratch_shapes=[pltpu.VMEM((K, D), jnp.float32)],
    )(x1, w1, w2, w3, g2, b2)


def _scatter_body(idx_ref, y_ref, o_in_ref, o_ref):
    del idx_ref, o_in_ref
    o_ref[0] = y_ref[0]


def _scatter(idx_flat, y2, out0_2d):
    grid_spec = pltpu.PrefetchScalarGridSpec(
        num_scalar_prefetch=1,
        grid=(B * K,),
        in_specs=[
            pl.BlockSpec((1, 1, D), lambda j, idx: (j, 0, 0)),
            pl.BlockSpec((1, 1, D), lambda j, idx: (idx[j], 0, 0)),
        ],
        out_specs=pl.BlockSpec((1, 1, D), lambda j, idx: (idx[j], 0, 0)),
    )
    return pl.pallas_call(
        _scatter_body,
        grid_spec=grid_spec,
        out_shape=jax.ShapeDtypeStruct((B * T, 1, D), jnp.float32),
        input_output_aliases={2: 0},
    )(idx_flat, y2.reshape(B * K, 1, D), out0_2d.reshape(B * T, 1, D))


def _gather(x2d, idx_flat):
    return _make_sc_gather()(x2d, idx_flat)


def kernel(x, w_router, ln1_g, ln1_b, ln2_g, ln2_b, in_proj_w, out_proj_w,
           w1, w2, w3):
    out0, scores = _score_copy(x, w_router)
    idxg, aux = _select(scores)
    idx_flat = idxg.reshape(B * K)
    x_sel = _gather(x.reshape(B * T, D), idx_flat)
    x_sel = x_sel.reshape(B, K, D)

    wq = in_proj_w[0:D]
    wk = in_proj_w[D:2 * D]
    wv = in_proj_w[2 * D:3 * D]
    x1 = _attn(x_sel, wq, wk, wv, out_proj_w.T,
               ln1_g.reshape(1, D), ln1_b.reshape(1, D))
    y = _ffn(x1, w1, w2, w3, ln2_g.reshape(1, D), ln2_b.reshape(1, D))

    out = _scatter(idx_flat, y.reshape(B * K, D), out0.reshape(B * T, D))
    return out.reshape(B, T, D), aux.reshape(())


# speedup vs baseline: 3.4048x; 3.4048x over previous
import functools
import math

import jax
import jax.numpy as jnp
from jax import lax
from jax.experimental import pallas as pl
from jax.experimental.pallas import tpu as pltpu
from jax.experimental.pallas import tpu_sc as plsc

D = 1024
H = 16
DH = 64
DFF = 4096
B = 4
T = 8192
K = 1024

TBLK = 2048
JCH = 256
FBLK = 1024

_NW = 32
_RPW = (B * K) // _NW
_GCH = 64


def _score_copy_body(x_ref, wr_ref, out_ref, s_ref):
    xb = x_ref[0]
    out_ref[0] = xb
    s = jnp.dot(xb, wr_ref[0], preferred_element_type=jnp.float32)
    s_ref[0, 0] = jax.nn.sigmoid(s)


def _score_copy(x, w_router):
    return pl.pallas_call(
        _score_copy_body,
        grid=(B, T // TBLK),
        in_specs=[
            pl.BlockSpec((1, TBLK, D), lambda b, t: (b, t, 0)),
            pl.BlockSpec((1, D), lambda b, t: (0, 0)),
        ],
        out_specs=[
            pl.BlockSpec((1, TBLK, D), lambda b, t: (b, t, 0)),
            pl.BlockSpec((1, 1, TBLK), lambda b, t: (b, 0, t)),
        ],
        out_shape=[
            jax.ShapeDtypeStruct((B, T, D), jnp.float32),
            jax.ShapeDtypeStruct((B, 1, T), jnp.float32),
        ],
    )(x, w_router)


def _select_body(s_ref, idx_ref, aux_ref):
    b = pl.program_id(0)
    c = pl.program_id(1)
    s = s_ref[:, 0, :]
    bits = lax.bitcast_convert_type(s, jnp.int32)

    lo = jnp.full((B, 1), -1, jnp.int32)
    hi = jnp.full((B, 1), 0x3F800000, jnp.int32)

    def bis(_, lh):
        lo, hi = lh
        mid = lax.div(lo + hi, 2)
        cnt = jnp.sum((bits > mid).astype(jnp.int32), axis=1, keepdims=True)
        take_hi = cnt < K
        return jnp.where(take_hi, lo, mid), jnp.where(take_hi, mid, hi)

    lo, hi = lax.fori_loop(0, 31, bis, (lo, hi))
    thr = hi
    gt = bits > thr
    eqm = bits == thr
    cgt = jnp.sum(gt.astype(jnp.int32), axis=1, keepdims=True)
    tneed = K - cgt
    pos = lax.broadcasted_iota(jnp.int32, (B, T), 1)

    lo2 = jnp.zeros((B, 1), jnp.int32)
    hi2 = jnp.full((B, 1), T, jnp.int32)

    def bis2(_, lh):
        lo, hi = lh
        mid = lax.div(lo + hi, 2)
        cnt = jnp.sum((eqm & (pos < mid)).astype(jnp.int32), axis=1,
                      keepdims=True)
        ok = cnt >= tneed
        return jnp.where(ok, lo, mid), jnp.where(ok, mid, hi)

    lo2, hi2 = lax.fori_loop(0, 13, bis2, (lo2, hi2))
    mask = gt | (eqm & (pos < hi2))
    mf = mask.astype(jnp.float32)

    mfr = mf.reshape(B * 64, 128)
    i128 = lax.broadcasted_iota(jnp.int32, (128, 128), 0)
    j128 = lax.broadcasted_iota(jnp.int32, (128, 128), 1)
    tri = (i128 <= j128).astype(jnp.float32)
    csum = jnp.dot(mfr, tri, preferred_element_type=jnp.float32)
    csum = csum.reshape(B, 64, 128)
    ssum = csum[:, :, 127]
    i64 = lax.broadcasted_iota(jnp.int32, (64, 64), 0)
    j64 = lax.broadcasted_iota(jnp.int32, (64, 64), 1)
    stri = (i64 < j64).astype(jnp.float32)
    off = jnp.dot(ssum, stri, preferred_element_type=jnp.float32)
    p = (csum + off[:, :, None]).reshape(B, T)

    rowm = lax.broadcasted_iota(jnp.int32, (B, T), 0) == b
    p_b = jnp.sum(jnp.where(rowm, p, 0.0), axis=0)
    m_b = jnp.sum(jnp.where(rowm & mask, 1.0, 0.0), axis=0)

    jvals = (lax.broadcasted_iota(jnp.int32, (1, JCH), 1) + c * JCH + 1
             ).astype(jnp.float32)
    onehot = ((p_b[:, None] == jvals) & (m_b[:, None] > 0.5)
              ).astype(jnp.float32)
    tv = lax.broadcasted_iota(jnp.int32, (1, T), 1).astype(jnp.float32)
    vals = jnp.dot(tv, onehot, preferred_element_type=jnp.float32)
    idx_ref[0, 0, :] = (vals[0] + b.astype(jnp.float32) * T).astype(jnp.int32)

    rs = jnp.sum(s, axis=1) / T
    mu = jnp.mean(rs)
    aux_ref[...] = (jnp.sum((rs - mu) ** 2) / (B - 1)).reshape(1, 1)


def _select(scores):
    return pl.pallas_call(
        _select_body,
        grid=(B, K // JCH),
        in_specs=[pl.BlockSpec((B, 1, T), lambda b, c: (0, 0, 0))],
        out_specs=[
            pl.BlockSpec((1, 1, JCH), lambda b, c: (b, 0, c)),
            pl.BlockSpec((1, 1), lambda b, c: (0, 0)),
        ],
        out_shape=[
            jax.ShapeDtypeStruct((B, 1, K), jnp.int32),
            jax.ShapeDtypeStruct((1, 1), jnp.float32),
        ],
    )(scores)


@functools.cache
def _make_sc_gather():
    mesh = plsc.VectorSubcoreMesh(core_axis_name="c", subcore_axis_name="s")

    @functools.partial(
        pl.kernel,
        out_type=jax.ShapeDtypeStruct((B * K, D), jnp.float32),
        mesh=mesh,
        scratch_types=[
            pltpu.VMEM((_GCH,), jnp.int32),
            pltpu.VMEM((_GCH, D), jnp.float32),
            pltpu.SemaphoreType.DMA,
        ],
    )
    def _sc_gather(x_hbm, idx_hbm, out_hbm, idx_v, rows_v, sem):
        wid = lax.axis_index("s") * 2 + lax.axis_index("c")
        base = wid * _RPW
        for ch in range(_RPW // _GCH):
            off = base + ch * _GCH
            pltpu.sync_copy(idx_hbm.at[pl.ds(off, _GCH)], idx_v)
            pltpu.async_copy(x_hbm.at[idx_v], rows_v, sem).wait()
            pltpu.sync_copy(rows_v, out_hbm.at[pl.ds(off, _GCH)])

    return _sc_gather


def _attn_body(xs_ref, wq_ref, wk_ref, wv_ref, wo_ref, g_ref, b_ref,
               out_ref, nrm_ref):
    h = pl.program_id(1)
    xb = xs_ref[0]

    @pl.when(h == 0)
    def _():
        mu = jnp.mean(xb, axis=1, keepdims=True)
        var = jnp.mean((xb - mu) ** 2, axis=1, keepdims=True)
        nrm_ref[...] = ((xb - mu) * lax.rsqrt(var + 1e-5) * g_ref[0]
                        + b_ref[0])

    normed = nrm_ref[...]
    cdim = (((1,), (1,)), ((), ()))
    q = lax.dot_general(normed, wq_ref[...], cdim,
                        preferred_element_type=jnp.float32)
    k = lax.dot_general(normed, wk_ref[...], cdim,
                        preferred_element_type=jnp.float32)
    v = lax.dot_general(normed, wv_ref[...], cdim,
                        preferred_element_type=jnp.float32)
    att = lax.dot_general(q, k, cdim,
                          preferred_element_type=jnp.float32) * (1.0 / 8.0)
    att = att - jnp.max(att, axis=1, keepdims=True)
    att = jnp.exp(att)
    att = att / jnp.sum(att, axis=1, keepdims=True)
    o = lax.dot_general(att, v, (((1,), (0,)), ((), ())),
                        preferred_element_type=jnp.float32)
    contrib = lax.dot_general(o, wo_ref[...], (((1,), (0,)), ((), ())),
                              preferred_element_type=jnp.float32)

    @pl.when(h == 0)
    def _():
        out_ref[0] = xb + contrib

    @pl.when(h != 0)
    def _():
        out_ref[0] = out_ref[0] + contrib


def _attn(x_sel, wq, wk, wv, wo, g1, b1):
    return pl.pallas_call(
        _attn_body,
        grid=(B, H),
        in_specs=[
            pl.BlockSpec((1, K, D), lambda b, h: (b, 0, 0)),
            pl.BlockSpec((DH, D), lambda b, h: (h, 0)),
            pl.BlockSpec((DH, D), lambda b, h: (h, 0)),
            pl.BlockSpec((DH, D), lambda b, h: (h, 0)),
            pl.BlockSpec((DH, D), lambda b, h: (h, 0)),
            pl.BlockSpec((1, D), lambda b, h: (0, 0)),
            pl.BlockSpec((1, D), lambda b, h: (0, 0)),
        ],
        out_specs=pl.BlockSpec((1, K, D), lambda b, h: (b, 0, 0)),
        out_shape=jax.ShapeDtypeStruct((B, K, D), jnp.float32),
        scratch_shapes=[pltpu.VMEM((K, D), jnp.float32)],
    )(x_sel, wq, wk, wv, wo, g1, b1)


def _ffn_body(x_ref, w1_ref, w2_ref, w3_ref, g_ref, b_ref, out_ref, h_ref):
    f = pl.program_id(1)
    xb = x_ref[0]

    @pl.when(f == 0)
    def _():
        mu = jnp.mean(xb, axis=1, keepdims=True)
        var = jnp.mean((xb - mu) ** 2, axis=1, keepdims=True)
        h_ref[...] = ((xb - mu) * lax.rsqrt(var + 1e-5) * g_ref[0]
                      + b_ref[0])

    hh = h_ref[...]
    cdim = (((1,), (1,)), ((), ()))
    a = lax.dot_general(hh, w1_ref[...], cdim,
                        preferred_element_type=jnp.float32)
    bb = lax.dot_general(hh, w2_ref[...], cdim,
                         preferred_element_type=jnp.float32)
    gg = (a * jax.nn.sigmoid(a)) * bb
    contrib = lax.dot_general(gg, w3_ref[...], cdim,
                              preferred_element_type=jnp.float32)

    @pl.when(f == 0)
    def _():
        out_ref[0] = xb + contrib

    @pl.when(f != 0)
    def _():
        out_ref[0] = out_ref[0] + contrib


def _ffn(x1, w1, w2, w3, g2, b2):
    return pl.pallas_call(
        _ffn_body,
        grid=(B, DFF // FBLK),
        in_specs=[
            pl.BlockSpec((1, K, D), lambda b, f: (b, 0, 0)),
            pl.BlockSpec((FBLK, D), lambda b, f: (f, 0)),
            pl.BlockSpec((FBLK, D), lambda b, f: (f, 0)),
            pl.BlockSpec((D, FBLK), lambda b, f: (0, f)),
            pl.BlockSpec((1, D), lambda b, f: (0, 0)),
            pl.BlockSpec((1, D), lambda b, f: (0, 0)),
        ],
        out_specs=pl.BlockSpec((1, K, D), lambda b, f: (b, 0, 0)),
        out_shape=jax.ShapeDtypeStruct((B, K, D), jnp.float32),
        scratch_shapes=[pltpu.VMEM((K, D), jnp.float32)],
    )(x1, w1, w2, w3, g2, b2)


@functools.cache
def _make_sc_scatter():
    mesh = plsc.VectorSubcoreMesh(core_axis_name="c", subcore_axis_name="s")

    @functools.partial(
        pl.kernel,
        out_type=(),
        mesh=mesh,
        scratch_types=[
            pltpu.VMEM((_GCH,), jnp.int32),
            pltpu.VMEM((_GCH, D), jnp.float32),
            pltpu.SemaphoreType.DMA,
        ],
    )
    def _sc_scatter(y_hbm, idx_hbm, out_ref, idx_v, rows_v, sem):
        wid = lax.axis_index("s") * 2 + lax.axis_index("c")
        base = wid * _RPW
        for ch in range(_RPW // _GCH):
            off = base + ch * _GCH
            pltpu.sync_copy(idx_hbm.at[pl.ds(off, _GCH)], idx_v)
            pltpu.sync_copy(y_hbm.at[pl.ds(off, _GCH)], rows_v)
            pltpu.async_copy(rows_v, out_ref.at[idx_v], sem).wait()

    return _sc_scatter


def _scatter(idx_flat, y2, out0_2d):
    out_ref = jax.new_ref(out0_2d)
    _make_sc_scatter()(y2, idx_flat, out_ref)
    return out_ref[...]


def _gather(x2d, idx_flat):
    return _make_sc_gather()(x2d, idx_flat)


def kernel(x, w_router, ln1_g, ln1_b, ln2_g, ln2_b, in_proj_w, out_proj_w,
           w1, w2, w3):
    out0, scores = _score_copy(x, w_router)
    idxg, aux = _select(scores)
    idx_flat = idxg.reshape(B * K)
    x_sel = _gather(x.reshape(B * T, D), idx_flat)
    x_sel = x_sel.reshape(B, K, D)

    wq = in_proj_w[0:D]
    wk = in_proj_w[D:2 * D]
    wv = in_proj_w[2 * D:3 * D]
    x1 = _attn(x_sel, wq, wk, wv, out_proj_w.T,
               ln1_g.reshape(1, D), ln1_b.reshape(1, D))
    y = _ffn(x1, w1, w2, w3, ln2_g.reshape(1, D), ln2_b.reshape(1, D))

    out = _scatter(idx_flat, y.reshape(B * K, D), out0.reshape(B * T, D))
    return out.reshape(B, T, D), aux.reshape(())


# speedup vs baseline: 4.4521x; 1.3076x over previous
import functools
import math

import jax
import jax.numpy as jnp
from jax import lax
from jax.experimental import pallas as pl
from jax.experimental.pallas import tpu as pltpu
from jax.experimental.pallas import tpu_sc as plsc

D = 1024
H = 16
DH = 64
DFF = 4096
B = 4
T = 8192
K = 1024

TBLK = 2048
JCH = 256
FBLK = 1024

_NW = 32
_RPW = (B * K) // _NW
_GCH = 64


def _score_copy_body(x_ref, wr_ref, out_ref, s_ref):
    xb = x_ref[0]
    out_ref[0] = xb
    s = jnp.dot(xb.astype(jnp.bfloat16), wr_ref[0].astype(jnp.bfloat16),
                preferred_element_type=jnp.float32)
    s_ref[0, 0] = jax.nn.sigmoid(s)


def _score_copy(x, w_router):
    return pl.pallas_call(
        _score_copy_body,
        grid=(B, T // TBLK),
        in_specs=[
            pl.BlockSpec((1, TBLK, D), lambda b, t: (b, t, 0)),
            pl.BlockSpec((1, D), lambda b, t: (0, 0)),
        ],
        out_specs=[
            pl.BlockSpec((1, TBLK, D), lambda b, t: (b, t, 0)),
            pl.BlockSpec((1, 1, TBLK), lambda b, t: (b, 0, t)),
        ],
        out_shape=[
            jax.ShapeDtypeStruct((B, T, D), jnp.float32),
            jax.ShapeDtypeStruct((B, 1, T), jnp.float32),
        ],
    )(x, w_router)


def _select_body(s_ref, idx_ref, aux_ref):
    s = s_ref[:, 0, :]
    bits = lax.bitcast_convert_type(s, jnp.int32)

    lo = jnp.full((B, 1), -1, jnp.int32)
    hi = jnp.full((B, 1), 0x3F800000, jnp.int32)

    def bis(_, lh):
        lo, hi = lh
        mid = lax.div(lo + hi, 2)
        cnt = jnp.sum((bits > mid).astype(jnp.int32), axis=1, keepdims=True)
        take_hi = cnt < K
        return jnp.where(take_hi, lo, mid), jnp.where(take_hi, mid, hi)

    lo, hi = lax.fori_loop(0, 31, bis, (lo, hi))
    thr = hi
    gt = bits > thr
    eqm = bits == thr
    cgt = jnp.sum(gt.astype(jnp.int32), axis=1, keepdims=True)
    tneed = K - cgt
    pos = lax.broadcasted_iota(jnp.int32, (B, T), 1)

    lo2 = jnp.zeros((B, 1), jnp.int32)
    hi2 = jnp.full((B, 1), T, jnp.int32)

    def bis2(_, lh):
        lo, hi = lh
        mid = lax.div(lo + hi, 2)
        cnt = jnp.sum((eqm & (pos < mid)).astype(jnp.int32), axis=1,
                      keepdims=True)
        ok = cnt >= tneed
        return jnp.where(ok, lo, mid), jnp.where(ok, mid, hi)

    lo2, hi2 = lax.fori_loop(0, 13, bis2, (lo2, hi2))
    mask = gt | (eqm & (pos < hi2))
    mf = mask.astype(jnp.float32)

    mfr = mf.reshape(B * 64, 128)
    i128 = lax.broadcasted_iota(jnp.int32, (128, 128), 0)
    j128 = lax.broadcasted_iota(jnp.int32, (128, 128), 1)
    tri = (i128 <= j128).astype(jnp.float32)
    csum = jnp.dot(mfr, tri, preferred_element_type=jnp.float32)
    csum = csum.reshape(B, 64, 128)
    ssum = csum[:, :, 127]
    i64 = lax.broadcasted_iota(jnp.int32, (64, 64), 0)
    j64 = lax.broadcasted_iota(jnp.int32, (64, 64), 1)
    stri = (i64 < j64).astype(jnp.float32)
    off = jnp.dot(ssum, stri, preferred_element_type=jnp.float32)
    p = (csum + off[:, :, None]).reshape(B, T)

    tvf = lax.broadcasted_iota(jnp.int32, (1, T), 1).astype(jnp.float32)
    nchunks = K // JCH
    for i in range(B * nchunks):
        b_i, c_i = divmod(i, nchunks)
        p_b = p[b_i:b_i + 1, :]
        m_b = mask[b_i:b_i + 1, :]
        jcol = (lax.broadcasted_iota(jnp.int32, (JCH, 1), 0)
                + (c_i * JCH + 1)).astype(jnp.float32)
        cond = (p_b == jcol) & m_b
        vals = jnp.sum(jnp.where(cond, tvf, 0.0), axis=1)
        idx_ref[i, 0, :] = (vals + float(b_i * T)).astype(jnp.int32)

    rs = jnp.sum(s, axis=1) / T
    mu = jnp.mean(rs)
    aux_ref[...] = (jnp.sum((rs - mu) ** 2) / (B - 1)).reshape(1, 1)


def _select(scores):
    return pl.pallas_call(
        _select_body,
        grid=(1,),
        in_specs=[pl.BlockSpec((B, 1, T), lambda g: (0, 0, 0))],
        out_specs=[
            pl.BlockSpec((B * K // JCH, 1, JCH), lambda g: (0, 0, 0)),
            pl.BlockSpec((1, 1), lambda g: (0, 0)),
        ],
        out_shape=[
            jax.ShapeDtypeStruct((B * K // JCH, 1, JCH), jnp.int32),
            jax.ShapeDtypeStruct((1, 1), jnp.float32),
        ],
    )(scores)


@functools.cache
def _make_sc_gather():
    mesh = plsc.VectorSubcoreMesh(core_axis_name="c", subcore_axis_name="s")

    @functools.partial(
        pl.kernel,
        out_type=jax.ShapeDtypeStruct((B * K, D), jnp.float32),
        mesh=mesh,
        scratch_types=[
            pltpu.VMEM((_GCH,), jnp.int32),
            pltpu.VMEM((_GCH, D), jnp.float32),
            pltpu.SemaphoreType.DMA,
        ],
    )
    def _sc_gather(x_hbm, idx_hbm, out_hbm, idx_v, rows_v, sem):
        wid = lax.axis_index("s") * 2 + lax.axis_index("c")
        base = wid * _RPW
        for ch in range(_RPW // _GCH):
            off = base + ch * _GCH
            pltpu.sync_copy(idx_hbm.at[pl.ds(off, _GCH)], idx_v)
            pltpu.async_copy(x_hbm.at[idx_v], rows_v, sem).wait()
            pltpu.sync_copy(rows_v, out_hbm.at[pl.ds(off, _GCH)])

    return _sc_gather


def _attn_body(xs_ref, wq_ref, wk_ref, wv_ref, wo_ref, g_ref, b_ref,
               out_ref, nrm_ref):
    h = pl.program_id(1)
    xb = xs_ref[0]

    @pl.when(h == 0)
    def _():
        mu = jnp.mean(xb, axis=1, keepdims=True)
        var = jnp.mean((xb - mu) ** 2, axis=1, keepdims=True)
        nrm_ref[...] = ((xb - mu) * lax.rsqrt(var + 1e-5) * g_ref[0]
                        + b_ref[0]).astype(jnp.bfloat16)

    normed = nrm_ref[...]
    cdim = (((1,), (1,)), ((), ()))
    q = lax.dot_general(normed, wq_ref[...], cdim,
                        preferred_element_type=jnp.float32)
    k = lax.dot_general(normed, wk_ref[...], cdim,
                        preferred_element_type=jnp.float32)
    v = lax.dot_general(normed, wv_ref[...], cdim,
                        preferred_element_type=jnp.float32)
    att = lax.dot_general(q.astype(jnp.bfloat16), k.astype(jnp.bfloat16),
                          cdim,
                          preferred_element_type=jnp.float32) * (1.0 / 8.0)
    att = att - jnp.max(att, axis=1, keepdims=True)
    att = jnp.exp(att)
    att = (att / jnp.sum(att, axis=1, keepdims=True)).astype(jnp.bfloat16)
    o = lax.dot_general(att, v.astype(jnp.bfloat16),
                        (((1,), (0,)), ((), ())),
                        preferred_element_type=jnp.float32)
    contrib = lax.dot_general(o.astype(jnp.bfloat16), wo_ref[...],
                              (((1,), (0,)), ((), ())),
                              preferred_element_type=jnp.float32)

    @pl.when(h == 0)
    def _():
        out_ref[0] = xb + contrib

    @pl.when(h != 0)
    def _():
        out_ref[0] = out_ref[0] + contrib


def _attn(x_sel, wq, wk, wv, wo, g1, b1):
    return pl.pallas_call(
        _attn_body,
        grid=(B, H),
        in_specs=[
            pl.BlockSpec((1, K, D), lambda b, h: (b, 0, 0)),
            pl.BlockSpec((DH, D), lambda b, h: (h, 0)),
            pl.BlockSpec((DH, D), lambda b, h: (h, 0)),
            pl.BlockSpec((DH, D), lambda b, h: (h, 0)),
            pl.BlockSpec((DH, D), lambda b, h: (h, 0)),
            pl.BlockSpec((1, D), lambda b, h: (0, 0)),
            pl.BlockSpec((1, D), lambda b, h: (0, 0)),
        ],
        out_specs=pl.BlockSpec((1, K, D), lambda b, h: (b, 0, 0)),
        out_shape=jax.ShapeDtypeStruct((B, K, D), jnp.float32),
        scratch_shapes=[pltpu.VMEM((K, D), jnp.bfloat16)],
    )(x_sel, wq, wk, wv, wo, g1, b1)


def _ffn_body(x_ref, w1_ref, w2_ref, w3_ref, g_ref, b_ref, out_ref, h_ref):
    f = pl.program_id(1)
    xb = x_ref[0]

    @pl.when(f == 0)
    def _():
        mu = jnp.mean(xb, axis=1, keepdims=True)
        var = jnp.mean((xb - mu) ** 2, axis=1, keepdims=True)
        h_ref[...] = ((xb - mu) * lax.rsqrt(var + 1e-5) * g_ref[0]
                      + b_ref[0]).astype(jnp.bfloat16)

    hh = h_ref[...]
    cdim = (((1,), (1,)), ((), ()))
    a = lax.dot_general(hh, w1_ref[...], cdim,
                        preferred_element_type=jnp.float32)
    bb = lax.dot_general(hh, w2_ref[...], cdim,
                         preferred_element_type=jnp.float32)
    gg = ((a * jax.nn.sigmoid(a)) * bb).astype(jnp.bfloat16)
    contrib = lax.dot_general(gg, w3_ref[...], cdim,
                              preferred_element_type=jnp.float32)

    @pl.when(f == 0)
    def _():
        out_ref[0] = xb + contrib

    @pl.when(f != 0)
    def _():
        out_ref[0] = out_ref[0] + contrib


def _ffn(x1, w1, w2, w3, g2, b2):
    return pl.pallas_call(
        _ffn_body,
        grid=(B, DFF // FBLK),
        in_specs=[
            pl.BlockSpec((1, K, D), lambda b, f: (b, 0, 0)),
            pl.BlockSpec((FBLK, D), lambda b, f: (f, 0)),
            pl.BlockSpec((FBLK, D), lambda b, f: (f, 0)),
            pl.BlockSpec((D, FBLK), lambda b, f: (0, f)),
            pl.BlockSpec((1, D), lambda b, f: (0, 0)),
            pl.BlockSpec((1, D), lambda b, f: (0, 0)),
        ],
        out_specs=pl.BlockSpec((1, K, D), lambda b, f: (b, 0, 0)),
        out_shape=jax.ShapeDtypeStruct((B, K, D), jnp.float32),
        scratch_shapes=[pltpu.VMEM((K, D), jnp.bfloat16)],
    )(x1, w1, w2, w3, g2, b2)


@functools.cache
def _make_sc_scatter():
    mesh = plsc.VectorSubcoreMesh(core_axis_name="c", subcore_axis_name="s")

    @functools.partial(
        pl.kernel,
        out_type=(),
        mesh=mesh,
        scratch_types=[
            pltpu.VMEM((_GCH,), jnp.int32),
            pltpu.VMEM((_GCH, D), jnp.float32),
            pltpu.SemaphoreType.DMA,
        ],
    )
    def _sc_scatter(y_hbm, idx_hbm, out_ref, idx_v, rows_v, sem):
        wid = lax.axis_index("s") * 2 + lax.axis_index("c")
        base = wid * _RPW
        for ch in range(_RPW // _GCH):
            off = base + ch * _GCH
            pltpu.sync_copy(idx_hbm.at[pl.ds(off, _GCH)], idx_v)
            pltpu.sync_copy(y_hbm.at[pl.ds(off, _GCH)], rows_v)
            pltpu.async_copy(rows_v, out_ref.at[idx_v], sem).wait()

    return _sc_scatter


def _scatter(idx_flat, y2, out0_2d):
    out_ref = jax.new_ref(out0_2d)
    _make_sc_scatter()(y2, idx_flat, out_ref)
    return out_ref[...]


def _gather(x2d, idx_flat):
    return _make_sc_gather()(x2d, idx_flat)


def kernel(x, w_router, ln1_g, ln1_b, ln2_g, ln2_b, in_proj_w, out_proj_w,
           w1, w2, w3):
    out0, scores = _score_copy(x, w_router)
    idxg, aux = _select(scores)
    idx_flat = idxg.reshape(B * K)
    x_sel = _gather(x.reshape(B * T, D), idx_flat)
    x_sel = x_sel.reshape(B, K, D)

    bf = jnp.bfloat16
    wq = in_proj_w[0:D].astype(bf)
    wk = in_proj_w[D:2 * D].astype(bf)
    wv = in_proj_w[2 * D:3 * D].astype(bf)
    x1 = _attn(x_sel, wq, wk, wv, out_proj_w.T.astype(bf),
               ln1_g.reshape(1, D), ln1_b.reshape(1, D))
    y = _ffn(x1, w1.astype(bf), w2.astype(bf), w3.astype(bf),
             ln2_g.reshape(1, D), ln2_b.reshape(1, D))

    out = _scatter(idx_flat, y.reshape(B * K, D), out0.reshape(B * T, D))
    return out.reshape(B, T, D), aux.reshape(())


# speedup vs baseline: 6.7092x; 1.5070x over previous
import functools
import math

import jax
import jax.numpy as jnp
from jax import lax
from jax.experimental import pallas as pl
from jax.experimental.pallas import tpu as pltpu
from jax.experimental.pallas import tpu_sc as plsc

D = 1024
H = 16
DH = 64
DFF = 4096
B = 4
T = 8192
K = 1024

TBLK = 2048
JCH = 256
FBLK = 1024

_NW = 32
_RPW = (B * K) // _NW
_GCH = 64


def _score_copy_body(x_ref, wr_ref, out_ref, s_ref):
    xb = x_ref[0]
    out_ref[0] = xb
    s = jnp.dot(xb.astype(jnp.bfloat16), wr_ref[0].astype(jnp.bfloat16),
                preferred_element_type=jnp.float32)
    s_ref[0, 0] = jax.nn.sigmoid(s)


def _score_copy(x, w_router):
    return pl.pallas_call(
        _score_copy_body,
        grid=(B, T // TBLK),
        in_specs=[
            pl.BlockSpec((1, TBLK, D), lambda b, t: (b, t, 0)),
            pl.BlockSpec((1, D), lambda b, t: (0, 0)),
        ],
        out_specs=[
            pl.BlockSpec((1, TBLK, D), lambda b, t: (b, t, 0)),
            pl.BlockSpec((1, 1, TBLK), lambda b, t: (b, 0, t)),
        ],
        out_shape=[
            jax.ShapeDtypeStruct((B, T, D), jnp.float32),
            jax.ShapeDtypeStruct((B, 1, T), jnp.float32),
        ],
    )(x, w_router)


def _select_body(s_ref, idx_ref, aux_ref):
    s = s_ref[:, 0, :]
    bits = lax.bitcast_convert_type(s, jnp.int32)

    lo = jnp.full((B, 1), -1, jnp.int32)
    hi = jnp.full((B, 1), 0x3F800000, jnp.int32)

    def bis(_, lh):
        lo, hi = lh
        mid = lax.div(lo + hi, 2)
        cnt = jnp.sum((bits > mid).astype(jnp.int32), axis=1, keepdims=True)
        take_hi = cnt < K
        return jnp.where(take_hi, lo, mid), jnp.where(take_hi, mid, hi)

    lo, hi = lax.fori_loop(0, 31, bis, (lo, hi))
    thr = hi
    gt = bits > thr
    eqm = bits == thr
    cgt = jnp.sum(gt.astype(jnp.int32), axis=1, keepdims=True)
    tneed = K - cgt
    pos = lax.broadcasted_iota(jnp.int32, (B, T), 1)

    lo2 = jnp.zeros((B, 1), jnp.int32)
    hi2 = jnp.full((B, 1), T, jnp.int32)

    def bis2(_, lh):
        lo, hi = lh
        mid = lax.div(lo + hi, 2)
        cnt = jnp.sum((eqm & (pos < mid)).astype(jnp.int32), axis=1,
                      keepdims=True)
        ok = cnt >= tneed
        return jnp.where(ok, lo, mid), jnp.where(ok, mid, hi)

    lo2, hi2 = lax.fori_loop(0, 13, bis2, (lo2, hi2))
    mask = gt | (eqm & (pos < hi2))
    mf = mask.astype(jnp.float32)

    mfr = mf.reshape(B * 64, 128)
    i128 = lax.broadcasted_iota(jnp.int32, (128, 128), 0)
    j128 = lax.broadcasted_iota(jnp.int32, (128, 128), 1)
    tri = (i128 <= j128).astype(jnp.float32)
    csum = jnp.dot(mfr, tri, preferred_element_type=jnp.float32)
    csum = csum.reshape(B, 64, 128)
    ssum = csum[:, :, 127]
    i64 = lax.broadcasted_iota(jnp.int32, (64, 64), 0)
    j64 = lax.broadcasted_iota(jnp.int32, (64, 64), 1)
    stri = (i64 < j64).astype(jnp.float32)
    off = jnp.dot(ssum, stri, preferred_element_type=jnp.float32)
    p = (csum + off[:, :, None]).reshape(B, T)

    tvf = lax.broadcasted_iota(jnp.int32, (1, T), 1).astype(jnp.float32)
    nchunks = K // JCH
    for i in range(B * nchunks):
        b_i, c_i = divmod(i, nchunks)
        p_b = p[b_i:b_i + 1, :]
        m_b = mask[b_i:b_i + 1, :]
        jcol = (lax.broadcasted_iota(jnp.int32, (JCH, 1), 0)
                + (c_i * JCH + 1)).astype(jnp.float32)
        cond = (p_b == jcol) & m_b
        vals = jnp.sum(jnp.where(cond, tvf, 0.0), axis=1)
        idx_ref[i, 0, :] = (vals + float(b_i * T)).astype(jnp.int32)

    rs = jnp.sum(s, axis=1) / T
    mu = jnp.mean(rs)
    aux_ref[...] = (jnp.sum((rs - mu) ** 2) / (B - 1)).reshape(1, 1)


def _select(scores):
    return pl.pallas_call(
        _select_body,
        grid=(1,),
        in_specs=[pl.BlockSpec((B, 1, T), lambda g: (0, 0, 0))],
        out_specs=[
            pl.BlockSpec((B * K // JCH, 1, JCH), lambda g: (0, 0, 0)),
            pl.BlockSpec((1, 1), lambda g: (0, 0)),
        ],
        out_shape=[
            jax.ShapeDtypeStruct((B * K // JCH, 1, JCH), jnp.int32),
            jax.ShapeDtypeStruct((1, 1), jnp.float32),
        ],
    )(scores)


@functools.cache
def _make_sc_gather():
    mesh = plsc.VectorSubcoreMesh(core_axis_name="c", subcore_axis_name="s")

    @functools.partial(
        pl.kernel,
        out_type=jax.ShapeDtypeStruct((B * K, D), jnp.float32),
        mesh=mesh,
        scratch_types=[
            pltpu.VMEM((_GCH,), jnp.int32),
            pltpu.VMEM((_GCH, D), jnp.float32),
            pltpu.SemaphoreType.DMA,
        ],
    )
    def _sc_gather(x_hbm, idx_hbm, out_hbm, idx_v, rows_v, sem):
        wid = lax.axis_index("s") * 2 + lax.axis_index("c")
        base = wid * _RPW
        for ch in range(_RPW // _GCH):
            off = base + ch * _GCH
            pltpu.sync_copy(idx_hbm.at[pl.ds(off, _GCH)], idx_v)
            pltpu.async_copy(x_hbm.at[idx_v], rows_v, sem).wait()
            pltpu.sync_copy(rows_v, out_hbm.at[pl.ds(off, _GCH)])

    return _sc_gather


def _qkv_body(xs_ref, w_ref, g_ref, b_ref, qkv_ref):
    xb = xs_ref[0]
    mu = jnp.mean(xb, axis=1, keepdims=True)
    var = jnp.mean((xb - mu) ** 2, axis=1, keepdims=True)
    nb = ((xb - mu) * lax.rsqrt(var + 1e-5) * g_ref[0]
          + b_ref[0]).astype(jnp.bfloat16)
    qkv = lax.dot_general(nb, w_ref[...], (((1,), (1,)), ((), ())),
                          preferred_element_type=jnp.float32)
    qkv_ref[0] = qkv.astype(jnp.bfloat16)


def _qkv(x_sel, w_in, g1, b1):
    return pl.pallas_call(
        _qkv_body,
        grid=(B,),
        in_specs=[
            pl.BlockSpec((1, K, D), lambda b: (b, 0, 0)),
            pl.BlockSpec((3 * D, D), lambda b: (0, 0)),
            pl.BlockSpec((1, D), lambda b: (0, 0)),
            pl.BlockSpec((1, D), lambda b: (0, 0)),
        ],
        out_specs=pl.BlockSpec((1, K, 3 * D), lambda b: (b, 0, 0)),
        out_shape=jax.ShapeDtypeStruct((B, K, 3 * D), jnp.bfloat16),
    )(x_sel, w_in, g1, b1)


def _att_body(q_ref, k_ref, v_ref, o_ref):
    q2 = q_ref[0] * jnp.bfloat16(0.125)
    k2 = k_ref[0]
    v2 = v_ref[0]
    halves = []
    for s in (0, 1):
        qa = q2[:, s * DH:(s + 1) * DH]
        ka = k2[:, s * DH:(s + 1) * DH]
        va = v2[:, s * DH:(s + 1) * DH]
        la = lax.dot_general(qa, ka, (((1,), (1,)), ((), ())),
                             preferred_element_type=jnp.float32)
        ea = jnp.exp(la)
        ra = 1.0 / jnp.sum(ea, axis=1, keepdims=True)
        oa = lax.dot_general(ea.astype(jnp.bfloat16), va,
                             (((1,), (0,)), ((), ())),
                             preferred_element_type=jnp.float32)
        halves.append((oa * ra).astype(jnp.bfloat16))
    o_ref[0] = jnp.concatenate(halves, axis=1)


def _att(qkv):
    return pl.pallas_call(
        _att_body,
        grid=(B, H // 2),
        in_specs=[
            pl.BlockSpec((1, K, 2 * DH), lambda b, h: (b, 0, h)),
            pl.BlockSpec((1, K, 2 * DH), lambda b, h: (b, 0, (H // 2) + h)),
            pl.BlockSpec((1, K, 2 * DH), lambda b, h: (b, 0, H + h)),
        ],
        out_specs=pl.BlockSpec((1, K, 2 * DH), lambda b, h: (b, 0, h)),
        out_shape=jax.ShapeDtypeStruct((B, K, D), jnp.bfloat16),
    )(qkv, qkv, qkv)


def _ffn_body(x_ref, o_ref, wo_ref, w1_ref, w2_ref, w3_ref, g_ref, b_ref,
              out_ref, x1_ref, h_ref):
    f = pl.program_id(1)
    cdim = (((1,), (1,)), ((), ()))

    @pl.when(f == 0)
    def _():
        xb = x_ref[0]
        proj = lax.dot_general(o_ref[0], wo_ref[...], (((1,), (0,)), ((), ())),
                               preferred_element_type=jnp.float32)
        x1 = xb + proj
        x1_ref[...] = x1
        mu = jnp.mean(x1, axis=1, keepdims=True)
        var = jnp.mean((x1 - mu) ** 2, axis=1, keepdims=True)
        h_ref[...] = ((x1 - mu) * lax.rsqrt(var + 1e-5) * g_ref[0]
                      + b_ref[0]).astype(jnp.bfloat16)

    hh = h_ref[...]
    a = lax.dot_general(hh, w1_ref[...], cdim,
                        preferred_element_type=jnp.float32)
    bb = lax.dot_general(hh, w2_ref[...], cdim,
                         preferred_element_type=jnp.float32)
    gg = ((a * jax.nn.sigmoid(a)) * bb).astype(jnp.bfloat16)
    contrib = lax.dot_general(gg, w3_ref[...], cdim,
                              preferred_element_type=jnp.float32)

    @pl.when(f == 0)
    def _():
        out_ref[0] = x1_ref[...] + contrib

    @pl.when(f != 0)
    def _():
        out_ref[0] = out_ref[0] + contrib


def _ffn(x_sel, o_full, wo_t, w1, w2, w3, g2, b2):
    return pl.pallas_call(
        _ffn_body,
        grid=(B, DFF // FBLK),
        in_specs=[
            pl.BlockSpec((1, K, D), lambda b, f: (b, 0, 0)),
            pl.BlockSpec((1, K, D), lambda b, f: (b, 0, 0)),
            pl.BlockSpec((D, D), lambda b, f: (0, 0)),
            pl.BlockSpec((FBLK, D), lambda b, f: (f, 0)),
            pl.BlockSpec((FBLK, D), lambda b, f: (f, 0)),
            pl.BlockSpec((D, FBLK), lambda b, f: (0, f)),
            pl.BlockSpec((1, D), lambda b, f: (0, 0)),
            pl.BlockSpec((1, D), lambda b, f: (0, 0)),
        ],
        out_specs=pl.BlockSpec((1, K, D), lambda b, f: (b, 0, 0)),
        out_shape=jax.ShapeDtypeStruct((B, K, D), jnp.float32),
        scratch_shapes=[pltpu.VMEM((K, D), jnp.float32),
                        pltpu.VMEM((K, D), jnp.bfloat16)],
    )(x_sel, o_full, wo_t, w1, w2, w3, g2, b2)


@functools.cache
def _make_sc_scatter():
    mesh = plsc.VectorSubcoreMesh(core_axis_name="c", subcore_axis_name="s")

    @functools.partial(
        pl.kernel,
        out_type=(),
        mesh=mesh,
        scratch_types=[
            pltpu.VMEM((_GCH,), jnp.int32),
            pltpu.VMEM((_GCH, D), jnp.float32),
            pltpu.SemaphoreType.DMA,
        ],
    )
    def _sc_scatter(y_hbm, idx_hbm, out_ref, idx_v, rows_v, sem):
        wid = lax.axis_index("s") * 2 + lax.axis_index("c")
        base = wid * _RPW
        for ch in range(_RPW // _GCH):
            off = base + ch * _GCH
            pltpu.sync_copy(idx_hbm.at[pl.ds(off, _GCH)], idx_v)
            pltpu.sync_copy(y_hbm.at[pl.ds(off, _GCH)], rows_v)
            pltpu.async_copy(rows_v, out_ref.at[idx_v], sem).wait()

    return _sc_scatter


def _scatter(idx_flat, y2, out0_2d):
    out_ref = jax.new_ref(out0_2d)
    _make_sc_scatter()(y2, idx_flat, out_ref)
    return out_ref[...]


def _gather(x2d, idx_flat):
    return _make_sc_gather()(x2d, idx_flat)


def kernel(x, w_router, ln1_g, ln1_b, ln2_g, ln2_b, in_proj_w, out_proj_w,
           w1, w2, w3):
    out0, scores = _score_copy(x, w_router)
    idxg, aux = _select(scores)
    idx_flat = idxg.reshape(B * K)
    x_sel = _gather(x.reshape(B * T, D), idx_flat)
    x_sel = x_sel.reshape(B, K, D)

    bf = jnp.bfloat16
    qkv = _qkv(x_sel, in_proj_w.astype(bf),
               ln1_g.reshape(1, D), ln1_b.reshape(1, D))
    o_full = _att(qkv)
    y = _ffn(x_sel, o_full, out_proj_w.T.astype(bf),
             w1.astype(bf), w2.astype(bf), w3.astype(bf),
             ln2_g.reshape(1, D), ln2_b.reshape(1, D))

    out = _scatter(idx_flat, y.reshape(B * K, D), out0.reshape(B * T, D))
    return out.reshape(B, T, D), aux.reshape(())


# speedup vs baseline: 6.7917x; 1.0123x over previous
import functools
import math

import jax
import jax.numpy as jnp
from jax import lax
from jax.experimental import pallas as pl
from jax.experimental.pallas import tpu as pltpu
from jax.experimental.pallas import tpu_sc as plsc

D = 1024
H = 16
DH = 64
DFF = 4096
B = 4
T = 8192
K = 1024

TBLK = 2048
JCH = 256
FBLK = 1024

_NW = 32
_RPW = (B * K) // _NW
_GCH = 64


def _score_copy_body(x_ref, wr_ref, out_ref, s_ref):
    xb = x_ref[0]
    out_ref[0] = xb
    s = jnp.dot(xb.astype(jnp.bfloat16), wr_ref[0].astype(jnp.bfloat16),
                preferred_element_type=jnp.float32)
    s_ref[0, 0] = jax.nn.sigmoid(s)


def _score_copy(x, w_router):
    return pl.pallas_call(
        _score_copy_body,
        grid=(B, T // TBLK),
        in_specs=[
            pl.BlockSpec((1, TBLK, D), lambda b, t: (b, t, 0)),
            pl.BlockSpec((1, D), lambda b, t: (0, 0)),
        ],
        out_specs=[
            pl.BlockSpec((1, TBLK, D), lambda b, t: (b, t, 0)),
            pl.BlockSpec((1, 1, TBLK), lambda b, t: (b, 0, t)),
        ],
        out_shape=[
            jax.ShapeDtypeStruct((B, T, D), jnp.float32),
            jax.ShapeDtypeStruct((B, 1, T), jnp.float32),
        ],
    )(x, w_router)


def _select_body(s_ref, idx_ref, aux_ref):
    s = s_ref[:, 0, :]
    bits = lax.bitcast_convert_type(s, jnp.int32)

    lo = jnp.full((B, 1), -1, jnp.int32)
    hi = jnp.full((B, 1), 0x3F800000, jnp.int32)

    def bis(_, lh):
        lo, hi = lh
        mid = lax.div(lo + hi, 2)
        cnt = jnp.sum((bits > mid).astype(jnp.int32), axis=1, keepdims=True)
        take_hi = cnt < K
        return jnp.where(take_hi, lo, mid), jnp.where(take_hi, mid, hi)

    lo, hi = lax.fori_loop(0, 31, bis, (lo, hi))
    thr = hi
    gt = bits > thr
    eqm = bits == thr
    cgt = jnp.sum(gt.astype(jnp.int32), axis=1, keepdims=True)
    tneed = K - cgt
    pos = lax.broadcasted_iota(jnp.int32, (B, T), 1)

    lo2 = jnp.zeros((B, 1), jnp.int32)
    hi2 = jnp.full((B, 1), T, jnp.int32)

    def bis2(_, lh):
        lo, hi = lh
        mid = lax.div(lo + hi, 2)
        cnt = jnp.sum((eqm & (pos < mid)).astype(jnp.int32), axis=1,
                      keepdims=True)
        ok = cnt >= tneed
        return jnp.where(ok, lo, mid), jnp.where(ok, mid, hi)

    lo2, hi2 = lax.fori_loop(0, 13, bis2, (lo2, hi2))
    mask = gt | (eqm & (pos < hi2))
    mf = mask.astype(jnp.float32)

    mfr = mf.reshape(B * 64, 128)
    i128 = lax.broadcasted_iota(jnp.int32, (128, 128), 0)
    j128 = lax.broadcasted_iota(jnp.int32, (128, 128), 1)
    tri = (i128 <= j128).astype(jnp.float32)
    csum = jnp.dot(mfr, tri, preferred_element_type=jnp.float32)
    csum = csum.reshape(B, 64, 128)
    ssum = csum[:, :, 127]
    i64 = lax.broadcasted_iota(jnp.int32, (64, 64), 0)
    j64 = lax.broadcasted_iota(jnp.int32, (64, 64), 1)
    stri = (i64 < j64).astype(jnp.float32)
    off = jnp.dot(ssum, stri, preferred_element_type=jnp.float32)
    p = (csum + off[:, :, None]).reshape(B, T)

    tvf = lax.broadcasted_iota(jnp.int32, (1, T), 1).astype(jnp.float32)
    nchunks = K // JCH
    for i in range(B * nchunks):
        b_i, c_i = divmod(i, nchunks)
        p_b = p[b_i:b_i + 1, :]
        m_b = mask[b_i:b_i + 1, :]
        jcol = (lax.broadcasted_iota(jnp.int32, (JCH, 1), 0)
                + (c_i * JCH + 1)).astype(jnp.float32)
        cond = (p_b == jcol) & m_b
        vals = jnp.sum(jnp.where(cond, tvf, 0.0), axis=1)
        idx_ref[i, 0, :] = (vals + float(b_i * T)).astype(jnp.int32)

    rs = jnp.sum(s, axis=1) / T
    mu = jnp.mean(rs)
    aux_ref[...] = (jnp.sum((rs - mu) ** 2) / (B - 1)).reshape(1, 1)


def _select(scores):
    return pl.pallas_call(
        _select_body,
        grid=(1,),
        in_specs=[pl.BlockSpec((B, 1, T), lambda g: (0, 0, 0))],
        out_specs=[
            pl.BlockSpec((B * K // JCH, 1, JCH), lambda g: (0, 0, 0)),
            pl.BlockSpec((1, 1), lambda g: (0, 0)),
        ],
        out_shape=[
            jax.ShapeDtypeStruct((B * K // JCH, 1, JCH), jnp.int32),
            jax.ShapeDtypeStruct((1, 1), jnp.float32),
        ],
    )(scores)


@functools.cache
def _make_sc_gather():
    mesh = plsc.VectorSubcoreMesh(core_axis_name="c", subcore_axis_name="s")

    @functools.partial(
        pl.kernel,
        out_type=jax.ShapeDtypeStruct((B * K, D), jnp.float32),
        mesh=mesh,
        scratch_types=[
            pltpu.VMEM((_GCH,), jnp.int32),
            pltpu.VMEM((_GCH, D), jnp.float32),
            pltpu.SemaphoreType.DMA,
        ],
    )
    def _sc_gather(x_hbm, idx_hbm, out_hbm, idx_v, rows_v, sem):
        wid = lax.axis_index("s") * 2 + lax.axis_index("c")
        base = wid * _RPW
        for ch in range(_RPW // _GCH):
            off = base + ch * _GCH
            pltpu.sync_copy(idx_hbm.at[pl.ds(off, _GCH)], idx_v)
            pltpu.async_copy(x_hbm.at[idx_v], rows_v, sem).wait()
            pltpu.sync_copy(rows_v, out_hbm.at[pl.ds(off, _GCH)])

    return _sc_gather


def _qkv_body(xs_ref, w_ref, g_ref, b_ref, qkv_ref):
    xb = xs_ref[0]
    mu = jnp.mean(xb, axis=1, keepdims=True)
    var = jnp.mean((xb - mu) ** 2, axis=1, keepdims=True)
    nb = ((xb - mu) * lax.rsqrt(var + 1e-5) * g_ref[0]
          + b_ref[0]).astype(jnp.bfloat16)
    qkv = lax.dot_general(nb, w_ref[...], (((1,), (1,)), ((), ())),
                          preferred_element_type=jnp.float32)
    qkv_ref[0] = qkv.astype(jnp.bfloat16)


def _qkv(x_sel, w_in, g1, b1):
    return pl.pallas_call(
        _qkv_body,
        grid=(B,),
        in_specs=[
            pl.BlockSpec((1, K, D), lambda b: (b, 0, 0)),
            pl.BlockSpec((3 * D, D), lambda b: (0, 0)),
            pl.BlockSpec((1, D), lambda b: (0, 0)),
            pl.BlockSpec((1, D), lambda b: (0, 0)),
        ],
        out_specs=pl.BlockSpec((1, K, 3 * D), lambda b: (b, 0, 0)),
        out_shape=jax.ShapeDtypeStruct((B, K, 3 * D), jnp.bfloat16),
    )(x_sel, w_in, g1, b1)


def _att_body(q_ref, k_ref, v_ref, o_ref):
    q2 = q_ref[0] * jnp.bfloat16(0.125)
    k2 = k_ref[0]
    v2 = v_ref[0]
    halves = []
    for s in (0, 1):
        qa = q2[:, s * DH:(s + 1) * DH]
        ka = k2[:, s * DH:(s + 1) * DH]
        va = v2[:, s * DH:(s + 1) * DH]
        la = lax.dot_general(qa, ka, (((1,), (1,)), ((), ())),
                             preferred_element_type=jnp.float32)
        ea = jnp.exp(la)
        ra = 1.0 / jnp.sum(ea, axis=1, keepdims=True)
        oa = lax.dot_general(ea.astype(jnp.bfloat16), va,
                             (((1,), (0,)), ((), ())),
                             preferred_element_type=jnp.float32)
        halves.append((oa * ra).astype(jnp.bfloat16))
    o_ref[0] = jnp.concatenate(halves, axis=1)


def _att(qkv):
    return pl.pallas_call(
        _att_body,
        grid=(B, H // 2),
        in_specs=[
            pl.BlockSpec((1, K, 2 * DH), lambda b, h: (b, 0, h)),
            pl.BlockSpec((1, K, 2 * DH), lambda b, h: (b, 0, (H // 2) + h)),
            pl.BlockSpec((1, K, 2 * DH), lambda b, h: (b, 0, H + h)),
        ],
        out_specs=pl.BlockSpec((1, K, 2 * DH), lambda b, h: (b, 0, h)),
        out_shape=jax.ShapeDtypeStruct((B, K, D), jnp.bfloat16),
    )(qkv, qkv, qkv)


def _ffn_body(x_ref, o_ref, wo_ref, w1_ref, w2_ref, w3_ref, g_ref, b_ref,
              out_ref, h_ref):
    f = pl.program_id(1)
    cdim = (((1,), (1,)), ((), ()))

    @pl.when(f == 0)
    def _():
        xb = x_ref[0]
        proj = lax.dot_general(o_ref[0], wo_ref[...], (((1,), (0,)), ((), ())),
                               preferred_element_type=jnp.float32)
        x1 = xb + proj
        out_ref[0] = x1
        mu = jnp.mean(x1, axis=1, keepdims=True)
        var = jnp.mean((x1 - mu) ** 2, axis=1, keepdims=True)
        h_ref[...] = ((x1 - mu) * lax.rsqrt(var + 1e-5) * g_ref[0]
                      + b_ref[0]).astype(jnp.bfloat16)

    hh = h_ref[...]
    contrib = None
    for c in range(2):
        w1b = w1_ref[c * FBLK:(c + 1) * FBLK, :]
        w2b = w2_ref[c * FBLK:(c + 1) * FBLK, :]
        w3b = w3_ref[:, c * FBLK:(c + 1) * FBLK]
        a = lax.dot_general(hh, w1b, cdim,
                            preferred_element_type=jnp.float32)
        bb = lax.dot_general(hh, w2b, cdim,
                             preferred_element_type=jnp.float32)
        gg = ((a * jax.nn.sigmoid(a)) * bb).astype(jnp.bfloat16)
        piece = lax.dot_general(gg, w3b, cdim,
                                preferred_element_type=jnp.float32)
        contrib = piece if contrib is None else contrib + piece

    out_ref[0] = out_ref[0] + contrib


def _ffn(x_sel, o_full, wo_t, w1, w2, w3, g2, b2):
    return pl.pallas_call(
        _ffn_body,
        grid=(B, DFF // (2 * FBLK)),
        in_specs=[
            pl.BlockSpec((1, K, D), lambda b, f: (b, 0, 0)),
            pl.BlockSpec((1, K, D), lambda b, f: (b, 0, 0)),
            pl.BlockSpec((D, D), lambda b, f: (0, 0)),
            pl.BlockSpec((2 * FBLK, D), lambda b, f: (f, 0)),
            pl.BlockSpec((2 * FBLK, D), lambda b, f: (f, 0)),
            pl.BlockSpec((D, 2 * FBLK), lambda b, f: (0, f)),
            pl.BlockSpec((1, D), lambda b, f: (0, 0)),
            pl.BlockSpec((1, D), lambda b, f: (0, 0)),
        ],
        out_specs=pl.BlockSpec((1, K, D), lambda b, f: (b, 0, 0)),
        out_shape=jax.ShapeDtypeStruct((B, K, D), jnp.float32),
        scratch_shapes=[pltpu.VMEM((K, D), jnp.bfloat16)],
        compiler_params=pltpu.CompilerParams(
            vmem_limit_bytes=100 * 1024 * 1024),
    )(x_sel, o_full, wo_t, w1, w2, w3, g2, b2)


@functools.cache
def _make_sc_scatter():
    mesh = plsc.VectorSubcoreMesh(core_axis_name="c", subcore_axis_name="s")

    @functools.partial(
        pl.kernel,
        out_type=(),
        mesh=mesh,
        scratch_types=[
            pltpu.VMEM((_GCH,), jnp.int32),
            pltpu.VMEM((_GCH, D), jnp.float32),
            pltpu.SemaphoreType.DMA,
        ],
    )
    def _sc_scatter(y_hbm, idx_hbm, out_ref, idx_v, rows_v, sem):
        wid = lax.axis_index("s") * 2 + lax.axis_index("c")
        base = wid * _RPW
        for ch in range(_RPW // _GCH):
            off = base + ch * _GCH
            pltpu.sync_copy(idx_hbm.at[pl.ds(off, _GCH)], idx_v)
            pltpu.sync_copy(y_hbm.at[pl.ds(off, _GCH)], rows_v)
            pltpu.async_copy(rows_v, out_ref.at[idx_v], sem).wait()

    return _sc_scatter


def _scatter(idx_flat, y2, out0_2d):
    out_ref = jax.new_ref(out0_2d)
    _make_sc_scatter()(y2, idx_flat, out_ref)
    return out_ref[...]


def _gather(x2d, idx_flat):
    return _make_sc_gather()(x2d, idx_flat)


def kernel(x, w_router, ln1_g, ln1_b, ln2_g, ln2_b, in_proj_w, out_proj_w,
           w1, w2, w3):
    out0, scores = _score_copy(x, w_router)
    idxg, aux = _select(scores)
    idx_flat = idxg.reshape(B * K)
    x_sel = _gather(x.reshape(B * T, D), idx_flat)
    x_sel = x_sel.reshape(B, K, D)

    bf = jnp.bfloat16
    qkv = _qkv(x_sel, in_proj_w.astype(bf),
               ln1_g.reshape(1, D), ln1_b.reshape(1, D))
    o_full = _att(qkv)
    y = _ffn(x_sel, o_full, out_proj_w.T.astype(bf),
             w1.astype(bf), w2.astype(bf), w3.astype(bf),
             ln2_g.reshape(1, D), ln2_b.reshape(1, D))

    out = _scatter(idx_flat, y.reshape(B * K, D), out0.reshape(B * T, D))
    return out.reshape(B, T, D), aux.reshape(())
